# Initial kernel scaffold; baseline (speedup 1.0000x reference)
import jax
import jax.numpy as jnp
from jax.experimental import pallas as pl


def kernel(x, edge_index, edge_type, W1, root1, b1, W2, root2, b2, mlp1_w, mlp1_b, mlp2_w, mlp2_b):
    raise NotImplementedError("write your pallas kernel here")


# speedup vs baseline: 4.8363x; 4.8363x over previous
import functools

import jax
import jax.numpy as jnp
from jax import lax
from jax.experimental import pallas as pl
from jax.experimental.pallas import tpu as pltpu
from jax.experimental.pallas import tpu_sc as plsc

_NC = 2
_NS = 16
_NW = _NC * _NS
_CH = 80
_ZCH = 104
_BN = 1000


def _sc_mesh():
    return plsc.VectorSubcoreMesh(core_axis_name="c", subcore_axis_name="s")


@functools.lru_cache(maxsize=None)
def _make_counts(N, E):
    EW = E // _NW
    nchunk = EW // _CH
    rpt = (N // _NS) // 8 * 8
    extra = N - _NS * rpt
    nz = rpt // _ZCH
    assert rpt % _ZCH == 0 and extra % 8 == 0 and extra <= _ZCH

    @functools.partial(
        pl.kernel, mesh=_sc_mesh(),
        out_type=jax.ShapeDtypeStruct((_NC, N, 128), jnp.float32),
        scratch_types=[
            pltpu.VMEM((_CH,), jnp.int32),
            pltpu.VMEM((_CH,), jnp.int32),
            pltpu.VMEM((_CH, 128), jnp.float32),
            pltpu.VMEM((_ZCH, 128), jnp.float32),
            pltpu.VMEM_SHARED((N, 128), jnp.float32),
            pltpu.SemaphoreType.DMA,
        ],
    )
    def counts_k(dst, etype, onehot, cntp, dst_v, typ_v, oh_v, z_v, cnt2, sem):
        cid = lax.axis_index("c")
        sid = lax.axis_index("s")
        wid = cid * _NS + sid

        def zrow(j, c):
            for k in range(8):
                z_v[j, pl.ds(k * 16, 16)] = jnp.zeros((16,), jnp.float32)
            return c
        lax.fori_loop(0, _ZCH, zrow, 0)
        base_r = pl.multiple_of(sid * rpt, 8)
        for m in range(nz):
            pltpu.sync_copy(z_v, cnt2.at[pl.ds(base_r + m * _ZCH, _ZCH)])
        if extra:
            @pl.when(sid == _NS - 1)
            def _():
                pltpu.sync_copy(z_v.at[pl.ds(0, extra)],
                                cnt2.at[pl.ds(_NS * rpt, extra)])
        plsc.subcore_barrier()

        ebase = wid * EW

        def chunk(c, carry):
            off = pl.multiple_of(ebase + c * _CH, 8)
            pltpu.sync_copy(dst.at[pl.ds(off, _CH)], dst_v)
            pltpu.sync_copy(etype.at[pl.ds(off, _CH)], typ_v)
            pltpu.async_copy(onehot.at[typ_v], oh_v, sem).wait()
            pltpu.sync_copy(oh_v, cnt2.at[dst_v], add=True)
            return carry
        lax.fori_loop(0, nchunk, chunk, 0)
        plsc.subcore_barrier()
        pltpu.sync_copy(cnt2.at[pl.ds(base_r, rpt)],
                        cntp.at[cid, pl.ds(base_r, rpt)])
        if extra:
            @pl.when(sid == _NS - 1)
            def _():
                pltpu.sync_copy(cnt2.at[pl.ds(_NS * rpt, extra)],
                                cntp.at[cid, pl.ds(_NS * rpt, extra)])

    return counts_k


@functools.lru_cache(maxsize=None)
def _make_weights(N, E):
    EW = E // _NW
    nchunk = EW // _CH

    @functools.partial(
        pl.kernel, mesh=_sc_mesh(),
        out_type=(jax.ShapeDtypeStruct((E, 16), jnp.float32),
                  jax.ShapeDtypeStruct((E,), jnp.int32)),
        scratch_types=[
            pltpu.VMEM((_CH,), jnp.int32),
            pltpu.VMEM((_CH + 16,), jnp.int32),
            pltpu.VMEM((_CH,), jnp.int32),
            pltpu.VMEM((_CH, 128), jnp.float32),
            pltpu.VMEM((_CH, 128), jnp.float32),
            pltpu.VMEM((_CH, 16), jnp.float32),
            pltpu.SemaphoreType.DMA,
            pltpu.SemaphoreType.DMA,
        ],
    )
    def weights_k(dst, etype, srcv, cnt0, cnt1, w16, gidx,
                  dst_v, typ_v, src_v, ra_v, rb_v, w_v, sem_a, sem_b):
        cid = lax.axis_index("c")
        sid = lax.axis_index("s")
        wid = cid * _NS + sid
        ebase = wid * EW

        def chunk(c, carry):
            off = pl.multiple_of(ebase + c * _CH, 8)
            pltpu.sync_copy(dst.at[pl.ds(off, _CH)], dst_v)
            pltpu.sync_copy(etype.at[pl.ds(off, _CH)], typ_v.at[pl.ds(0, _CH)])
            pltpu.sync_copy(srcv.at[pl.ds(off, _CH)], src_v)
            cp_a = pltpu.async_copy(cnt0.at[dst_v], ra_v, sem_a)
            cp_b = pltpu.async_copy(cnt1.at[dst_v], rb_v, sem_b)
            cp_a.wait()
            cp_b.wait()
            for g in range(_CH // 16):
                t = typ_v[pl.ds(g * 16, 16)]
                s = src_v[pl.ds(g * 16, 16)]
                src_v[pl.ds(g * 16, 16)] = t * N + s

            def ext(j, cc):
                b = typ_v[pl.ds(j, 16)][0]
                c16 = ra_v[j, pl.ds(b * 16, 16)] + rb_v[j, pl.ds(b * 16, 16)]
                w_v[j, pl.ds(0, 16)] = 1.0 / jnp.maximum(c16, 1.0)
                return cc
            lax.fori_loop(0, _CH, ext, 0)
            pltpu.sync_copy(w_v, w16.at[pl.ds(off, _CH)])
            pltpu.sync_copy(src_v, gidx.at[pl.ds(off, _CH)])
            return carry
        lax.fori_loop(0, nchunk, chunk, 0)

    return weights_k


@functools.lru_cache(maxsize=None)
def _make_edgepass(N, E, RN1):
    EW = E // _NW
    nchunk = EW // _CH
    rpt = (N // _NS) // 8 * 8
    extra = N - _NS * rpt
    nz = rpt // _ZCH
    assert rpt % _ZCH == 0 and extra % 8 == 0 and extra <= _ZCH

    @functools.partial(
        pl.kernel, mesh=_sc_mesh(),
        out_type=jax.ShapeDtypeStruct((_NC, N, 128), jnp.float32),
        scratch_types=[
            pltpu.VMEM((_CH,), jnp.int32),
            pltpu.VMEM((_CH,), jnp.int32),
            pltpu.VMEM((_CH, 16), jnp.float32),
            pltpu.VMEM((_CH, 128), jnp.float32),
            pltpu.VMEM((_ZCH, 128), jnp.float32),
            pltpu.VMEM_SHARED((N, 128), jnp.float32),
            pltpu.SemaphoreType.DMA,
        ],
    )
    def edge_k(yall, gidx, dst, w16, outp,
               gi_v, dst_v, w_v, rows_v, z_v, accum, sem):
        cid = lax.axis_index("c")
        sid = lax.axis_index("s")
        wid = cid * _NS + sid

        def zrow(j, c):
            for k in range(8):
                z_v[j, pl.ds(k * 16, 16)] = jnp.zeros((16,), jnp.float32)
            return c
        lax.fori_loop(0, _ZCH, zrow, 0)
        base_r = pl.multiple_of(sid * rpt, 8)
        for m in range(nz):
            pltpu.sync_copy(z_v, accum.at[pl.ds(base_r + m * _ZCH, _ZCH)])
        if extra:
            @pl.when(sid == _NS - 1)
            def _():
                pltpu.sync_copy(z_v.at[pl.ds(0, extra)],
                                accum.at[pl.ds(_NS * rpt, extra)])
        plsc.subcore_barrier()

        ebase = wid * EW

        def chunk(c, carry):
            off = pl.multiple_of(ebase + c * _CH, 8)
            pltpu.sync_copy(gidx.at[pl.ds(off, _CH)], gi_v)
            pltpu.sync_copy(dst.at[pl.ds(off, _CH)], dst_v)
            pltpu.sync_copy(w16.at[pl.ds(off, _CH)], w_v)
            pltpu.async_copy(yall.at[gi_v], rows_v, sem).wait()

            def srow(j, cc):
                wj = w_v[j, pl.ds(0, 16)]
                for k in range(8):
                    rows_v[j, pl.ds(k * 16, 16)] = rows_v[j, pl.ds(k * 16, 16)] * wj
                return cc
            lax.fori_loop(0, _CH, srow, 0)
            pltpu.sync_copy(rows_v, accum.at[dst_v], add=True)
            return carry
        lax.fori_loop(0, nchunk, chunk, 0)
        plsc.subcore_barrier()
        pltpu.sync_copy(accum.at[pl.ds(base_r, rpt)],
                        outp.at[cid, pl.ds(base_r, rpt)])
        if extra:
            @pl.when(sid == _NS - 1)
            def _():
                pltpu.sync_copy(accum.at[pl.ds(_NS * rpt, extra)],
                                outp.at[cid, pl.ds(_NS * rpt, extra)])

    return edge_k


def _transform(x, wall, ball):
    N, D = x.shape
    R1, _, H = wall.shape
    NB = N // _BN

    def body(x_ref, w_ref, b_ref, o_ref):
        o_ref[...] = (jnp.dot(x_ref[...], w_ref[0],
                              preferred_element_type=jnp.float32) + b_ref[0])

    return pl.pallas_call(
        body,
        grid=(R1, NB),
        in_specs=[
            pl.BlockSpec((_BN, D), lambda r, i: (i, 0)),
            pl.BlockSpec((1, D, H), lambda r, i: (r, 0, 0)),
            pl.BlockSpec((1, 1, H), lambda r, i: (r, 0, 0)),
        ],
        out_specs=pl.BlockSpec((_BN, H), lambda r, i: (r * NB + i, 0)),
        out_shape=jax.ShapeDtypeStruct((R1 * N, H), jnp.float32),
    )(x, wall, ball)


def _combine(yall, partials, N, H, R):
    NB = N // _BN

    def body(y_ref, p_ref, o_ref):
        o_ref[...] = jnp.maximum(y_ref[...] + p_ref[0] + p_ref[1], 0.0)

    return pl.pallas_call(
        body,
        grid=(NB,),
        in_specs=[
            pl.BlockSpec((_BN, H), lambda i: (R * NB + i, 0)),
            pl.BlockSpec((2, _BN, H), lambda i: (0, i, 0)),
        ],
        out_specs=pl.BlockSpec((_BN, H), lambda i: (i, 0)),
        out_shape=jax.ShapeDtypeStruct((N, H), jnp.float32),
    )(yall, partials)


def _mlp(h, w1, b1, w2, b2):
    N, H = h.shape
    C = w2.shape[1]
    NB = N // _BN
    b1r = b1.reshape(1, H)
    b2r = b2.reshape(1, C)

    def body(h_ref, w1_ref, b1_ref, w2_ref, b2_ref, lo_ref, po_ref):
        z = (jnp.dot(h_ref[...], w1_ref[...],
                     preferred_element_type=jnp.float32) + b1_ref[...])
        z = jnp.where(z > 0, z, jnp.exp(jnp.minimum(z, 0.0)) - 1.0)
        lg = (jnp.dot(z, w2_ref[...],
                      preferred_element_type=jnp.float32) + b2_ref[...])
        lo_ref[...] = lg
        m = jnp.max(lg, axis=-1, keepdims=True)
        e = jnp.exp(lg - m)
        po_ref[...] = e / jnp.sum(e, axis=-1, keepdims=True)

    return pl.pallas_call(
        body,
        grid=(NB,),
        in_specs=[
            pl.BlockSpec((_BN, H), lambda i: (i, 0)),
            pl.BlockSpec((H, H), lambda i: (0, 0)),
            pl.BlockSpec((1, H), lambda i: (0, 0)),
            pl.BlockSpec((H, C), lambda i: (0, 0)),
            pl.BlockSpec((1, C), lambda i: (0, 0)),
        ],
        out_specs=(pl.BlockSpec((_BN, C), lambda i: (i, 0)),
                   pl.BlockSpec((_BN, C), lambda i: (i, 0))),
        out_shape=(jax.ShapeDtypeStruct((N, C), jnp.float32),
                   jax.ShapeDtypeStruct((N, C), jnp.float32)),
    )(h, w1, b1r, w2, b2r)


def kernel(x, edge_index, edge_type, W1, root1, b1, W2, root2, b2,
           mlp1_w, mlp1_b, mlp2_w, mlp2_b):
    N, D = x.shape
    E = edge_type.shape[0]
    R = W1.shape[0]
    H = root1.shape[1]
    assert D == 128 and H == 128 and E % (_NW * _CH) == 0 and N % _NS == 0

    src = edge_index[0]
    dst = edge_index[1]
    onehot = (jax.lax.broadcasted_iota(jnp.int32, (R, 128), 1) // 16
              == jax.lax.broadcasted_iota(jnp.int32, (R, 128), 0)
              ).astype(jnp.float32)

    cntp = _make_counts(N, E)(dst, edge_type, onehot)
    w16, gidx = _make_weights(N, E)(dst, edge_type, src, cntp[0], cntp[1])

    edge_k = _make_edgepass(N, E, (R + 1) * N)

    wall1 = jnp.concatenate([W1, root1[None]], axis=0)
    ball1 = jnp.zeros((R + 1, 1, H), jnp.float32).at[R, 0].set(b1)
    yall1 = _transform(x, wall1, ball1)
    p1 = edge_k(yall1, gidx, dst, w16)
    h1 = _combine(yall1, p1, N, H, R)

    wall2 = jnp.concatenate([W2, root2[None]], axis=0)
    ball2 = jnp.zeros((R + 1, 1, H), jnp.float32).at[R, 0].set(b2)
    yall2 = _transform(h1, wall2, ball2)
    p2 = edge_k(yall2, gidx, dst, w16)
    h2 = _combine(yall2, p2, N, H, R)

    logits, probs = _mlp(h2, mlp1_w, mlp1_b, mlp2_w, mlp2_b)
    return (logits, probs, h2)


# speedup vs baseline: 8.5711x; 1.7722x over previous
import functools

import jax
import jax.numpy as jnp
from jax import lax
from jax.experimental import pallas as pl
from jax.experimental.pallas import tpu as pltpu
from jax.experimental.pallas import tpu_sc as plsc

_NC = 2
_NS = 16
_NW = _NC * _NS
_CH = 80
_ZCH = 104
_BN = 1000


def _sc_mesh():
    return plsc.VectorSubcoreMesh(core_axis_name="c", subcore_axis_name="s")


@functools.lru_cache(maxsize=None)
def _make_counts(N, E, R):
    EW = E // _NW
    nchunk = EW // _CH
    rpt = (N // _NS) // 8 * 8
    extra = N - _NS * rpt
    nz = rpt // _ZCH
    assert rpt % _ZCH == 0 and extra % 8 == 0 and extra <= _ZCH

    @functools.partial(
        pl.kernel, mesh=_sc_mesh(),
        out_type=jax.ShapeDtypeStruct((_NC, N, 128), jnp.float32),
        scratch_types=[
            pltpu.VMEM((_CH,), jnp.int32),
            pltpu.VMEM((_CH,), jnp.int32),
            pltpu.VMEM((_CH, 128), jnp.float32),
            pltpu.VMEM((_ZCH, 128), jnp.float32),
            pltpu.VMEM_SHARED((8, 128), jnp.float32),
            pltpu.VMEM_SHARED((N, 128), jnp.float32),
            pltpu.SemaphoreType.DMA,
        ],
    )
    def counts_k(dst, etype, onehot, cntp,
                 dst_v, typ_v, oh_v, z_v, ohs, cnt2, sem):
        cid = lax.axis_index("c")
        sid = lax.axis_index("s")
        wid = cid * _NS + sid

        def zrow(j, c):
            for k in range(8):
                z_v[j, pl.ds(k * 16, 16)] = jnp.zeros((16,), jnp.float32)
            return c
        lax.fori_loop(0, _ZCH, zrow, 0)
        base_r = pl.multiple_of(sid * rpt, 8)
        for m in range(nz):
            pltpu.sync_copy(z_v, cnt2.at[pl.ds(base_r + m * _ZCH, _ZCH)])
        if extra:
            @pl.when(sid == _NS - 1)
            def _():
                pltpu.sync_copy(z_v.at[pl.ds(0, extra)],
                                cnt2.at[pl.ds(_NS * rpt, extra)])

        @pl.when(sid == 0)
        def _():
            pltpu.sync_copy(onehot, ohs)
        plsc.subcore_barrier()

        ebase = wid * EW

        def chunk(c, carry):
            off = pl.multiple_of(ebase + c * _CH, 8)
            pltpu.sync_copy(dst.at[pl.ds(off, _CH)], dst_v)
            pltpu.sync_copy(etype.at[pl.ds(off, _CH)], typ_v)
            pltpu.async_copy(ohs.at[typ_v], oh_v, sem).wait()
            pltpu.sync_copy(oh_v, cnt2.at[dst_v], add=True)
            return carry
        lax.fori_loop(0, nchunk, chunk, 0)
        plsc.subcore_barrier()
        pltpu.sync_copy(cnt2.at[pl.ds(base_r, rpt)],
                        cntp.at[cid, pl.ds(base_r, rpt)])
        if extra:
            @pl.when(sid == _NS - 1)
            def _():
                pltpu.sync_copy(cnt2.at[pl.ds(_NS * rpt, extra)],
                                cntp.at[cid, pl.ds(_NS * rpt, extra)])

    return counts_k


def _weight_table(cntp, mhot, N, R):
    NB = N // _BN

    def body(p_ref, m_ref, o_ref):
        w = 1.0 / jnp.maximum(p_ref[0] + p_ref[1], 1.0)
        o_ref[...] = jnp.dot(w, m_ref[0], preferred_element_type=jnp.float32)

    return pl.pallas_call(
        body,
        grid=(R, NB),
        in_specs=[
            pl.BlockSpec((2, _BN, 128), lambda t, i: (0, i, 0)),
            pl.BlockSpec((1, 128, 128), lambda t, i: (t, 0, 0)),
        ],
        out_specs=pl.BlockSpec((_BN, 128), lambda t, i: (t * NB + i, 0)),
        out_shape=jax.ShapeDtypeStruct((R * N, 128), jnp.float32),
    )(cntp, mhot)


@functools.lru_cache(maxsize=None)
def _make_weights(N, E):
    EW = E // _NW
    nchunk = EW // _CH

    @functools.partial(
        pl.kernel, mesh=_sc_mesh(),
        out_type=(jax.ShapeDtypeStruct((E, 16), jnp.float32),
                  jax.ShapeDtypeStruct((E,), jnp.int32)),
        scratch_types=[
            pltpu.VMEM((_CH,), jnp.int32),
            pltpu.VMEM((_CH,), jnp.int32),
            pltpu.VMEM((_CH,), jnp.int32),
            pltpu.VMEM((_CH,), jnp.int32),
            pltpu.VMEM((_CH, 128), jnp.float32),
            pltpu.VMEM((_CH, 16), jnp.float32),
            pltpu.SemaphoreType.DMA,
        ],
    )
    def weights_k(dst, etype, srcv, wtab, w16, gidx,
                  dst_v, typ_v, src_v, g2_v, ra_v, w_v, sem):
        cid = lax.axis_index("c")
        sid = lax.axis_index("s")
        wid = cid * _NS + sid
        ebase = wid * EW

        def chunk(c, carry):
            off = pl.multiple_of(ebase + c * _CH, 8)
            pltpu.sync_copy(dst.at[pl.ds(off, _CH)], dst_v)
            pltpu.sync_copy(etype.at[pl.ds(off, _CH)], typ_v)
            pltpu.sync_copy(srcv.at[pl.ds(off, _CH)], src_v)
            for g in range(_CH // 16):
                t = typ_v[pl.ds(g * 16, 16)]
                g2_v[pl.ds(g * 16, 16)] = t * N + dst_v[pl.ds(g * 16, 16)]
                src_v[pl.ds(g * 16, 16)] = t * N + src_v[pl.ds(g * 16, 16)]
            pltpu.async_copy(wtab.at[g2_v], ra_v, sem).wait()

            def ext(j, cc):
                w_v[j, pl.ds(0, 16)] = ra_v[j, pl.ds(0, 16)]
                return cc
            lax.fori_loop(0, _CH, ext, 0)
            pltpu.sync_copy(w_v, w16.at[pl.ds(off, _CH)])
            pltpu.sync_copy(src_v, gidx.at[pl.ds(off, _CH)])
            return carry
        lax.fori_loop(0, nchunk, chunk, 0)

    return weights_k


@functools.lru_cache(maxsize=None)
def _make_edgepass(N, E, RN1):
    EW = E // _NW
    nchunk = EW // _CH
    rpt = (N // _NS) // 8 * 8
    extra = N - _NS * rpt
    nz = rpt // _ZCH
    assert rpt % _ZCH == 0 and extra % 8 == 0 and extra <= _ZCH

    @functools.partial(
        pl.kernel, mesh=_sc_mesh(),
        out_type=jax.ShapeDtypeStruct((_NC, N, 128), jnp.float32),
        scratch_types=[
            pltpu.VMEM((_CH,), jnp.int32),
            pltpu.VMEM((_CH,), jnp.int32),
            pltpu.VMEM((_CH, 16), jnp.float32),
            pltpu.VMEM((_CH, 128), jnp.float32),
            pltpu.VMEM((_ZCH, 128), jnp.float32),
            pltpu.VMEM_SHARED((N, 128), jnp.float32),
            pltpu.SemaphoreType.DMA,
        ],
    )
    def edge_k(yall, gidx, dst, w16, outp,
               gi_v, dst_v, w_v, rows_v, z_v, accum, sem):
        cid = lax.axis_index("c")
        sid = lax.axis_index("s")
        wid = cid * _NS + sid

        def zrow(j, c):
            for k in range(8):
                z_v[j, pl.ds(k * 16, 16)] = jnp.zeros((16,), jnp.float32)
            return c
        lax.fori_loop(0, _ZCH, zrow, 0)
        base_r = pl.multiple_of(sid * rpt, 8)
        for m in range(nz):
            pltpu.sync_copy(z_v, accum.at[pl.ds(base_r + m * _ZCH, _ZCH)])
        if extra:
            @pl.when(sid == _NS - 1)
            def _():
                pltpu.sync_copy(z_v.at[pl.ds(0, extra)],
                                accum.at[pl.ds(_NS * rpt, extra)])
        plsc.subcore_barrier()

        ebase = wid * EW

        def chunk(c, carry):
            off = pl.multiple_of(ebase + c * _CH, 8)
            pltpu.sync_copy(gidx.at[pl.ds(off, _CH)], gi_v)
            pltpu.sync_copy(dst.at[pl.ds(off, _CH)], dst_v)
            pltpu.sync_copy(w16.at[pl.ds(off, _CH)], w_v)
            pltpu.async_copy(yall.at[gi_v], rows_v, sem).wait()

            def srow(j, cc):
                wj = w_v[j, pl.ds(0, 16)]
                for k in range(8):
                    rows_v[j, pl.ds(k * 16, 16)] = rows_v[j, pl.ds(k * 16, 16)] * wj
                return cc
            lax.fori_loop(0, _CH, srow, 0)
            pltpu.sync_copy(rows_v, accum.at[dst_v], add=True)
            return carry
        lax.fori_loop(0, nchunk, chunk, 0)
        plsc.subcore_barrier()
        pltpu.sync_copy(accum.at[pl.ds(base_r, rpt)],
                        outp.at[cid, pl.ds(base_r, rpt)])
        if extra:
            @pl.when(sid == _NS - 1)
            def _():
                pltpu.sync_copy(accum.at[pl.ds(_NS * rpt, extra)],
                                outp.at[cid, pl.ds(_NS * rpt, extra)])

    return edge_k


def _transform(x, wall, ball):
    N, D = x.shape
    R1, _, H = wall.shape
    NB = N // _BN

    def body(x_ref, w_ref, b_ref, o_ref):
        o_ref[...] = (jnp.dot(x_ref[...], w_ref[0],
                              preferred_element_type=jnp.float32) + b_ref[0])

    return pl.pallas_call(
        body,
        grid=(R1, NB),
        in_specs=[
            pl.BlockSpec((_BN, D), lambda r, i: (i, 0)),
            pl.BlockSpec((1, D, H), lambda r, i: (r, 0, 0)),
            pl.BlockSpec((1, 1, H), lambda r, i: (r, 0, 0)),
        ],
        out_specs=pl.BlockSpec((_BN, H), lambda r, i: (r * NB + i, 0)),
        out_shape=jax.ShapeDtypeStruct((R1 * N, H), jnp.float32),
    )(x, wall, ball)


def _combine(yall, partials, N, H, R):
    NB = N // _BN

    def body(y_ref, p_ref, o_ref):
        o_ref[...] = jnp.maximum(y_ref[...] + p_ref[0] + p_ref[1], 0.0)

    return pl.pallas_call(
        body,
        grid=(NB,),
        in_specs=[
            pl.BlockSpec((_BN, H), lambda i: (R * NB + i, 0)),
            pl.BlockSpec((2, _BN, H), lambda i: (0, i, 0)),
        ],
        out_specs=pl.BlockSpec((_BN, H), lambda i: (i, 0)),
        out_shape=jax.ShapeDtypeStruct((N, H), jnp.float32),
    )(yall, partials)


def _mlp(h, w1, b1, w2, b2):
    N, H = h.shape
    C = w2.shape[1]
    NB = N // _BN
    b1r = b1.reshape(1, H)
    b2r = b2.reshape(1, C)

    def body(h_ref, w1_ref, b1_ref, w2_ref, b2_ref, lo_ref, po_ref):
        z = (jnp.dot(h_ref[...], w1_ref[...],
                     preferred_element_type=jnp.float32) + b1_ref[...])
        z = jnp.where(z > 0, z, jnp.exp(jnp.minimum(z, 0.0)) - 1.0)
        lg = (jnp.dot(z, w2_ref[...],
                      preferred_element_type=jnp.float32) + b2_ref[...])
        lo_ref[...] = lg
        m = jnp.max(lg, axis=-1, keepdims=True)
        e = jnp.exp(lg - m)
        po_ref[...] = e / jnp.sum(e, axis=-1, keepdims=True)

    return pl.pallas_call(
        body,
        grid=(NB,),
        in_specs=[
            pl.BlockSpec((_BN, H), lambda i: (i, 0)),
            pl.BlockSpec((H, H), lambda i: (0, 0)),
            pl.BlockSpec((1, H), lambda i: (0, 0)),
            pl.BlockSpec((H, C), lambda i: (0, 0)),
            pl.BlockSpec((1, C), lambda i: (0, 0)),
        ],
        out_specs=(pl.BlockSpec((_BN, C), lambda i: (i, 0)),
                   pl.BlockSpec((_BN, C), lambda i: (i, 0))),
        out_shape=(jax.ShapeDtypeStruct((N, C), jnp.float32),
                   jax.ShapeDtypeStruct((N, C), jnp.float32)),
    )(h, w1, b1r, w2, b2r)


def kernel(x, edge_index, edge_type, W1, root1, b1, W2, root2, b2,
           mlp1_w, mlp1_b, mlp2_w, mlp2_b):
    N, D = x.shape
    E = edge_type.shape[0]
    R = W1.shape[0]
    H = root1.shape[1]
    assert D == 128 and H == 128 and E % (_NW * _CH) == 0 and N % _NS == 0

    src = edge_index[0]
    dst = edge_index[1]
    onehot = (jax.lax.broadcasted_iota(jnp.int32, (R, 128), 1) // 16
              == jax.lax.broadcasted_iota(jnp.int32, (R, 128), 0)
              ).astype(jnp.float32)
    mhot = (jax.lax.broadcasted_iota(jnp.int32, (R, 128, 128), 1)
            == 16 * jax.lax.broadcasted_iota(jnp.int32, (R, 128, 128), 0)
            + jax.lax.broadcasted_iota(jnp.int32, (R, 128, 128), 2) % 16
            ).astype(jnp.float32)

    cntp = _make_counts(N, E, R)(dst, edge_type, onehot)
    wtab = _weight_table(cntp, mhot, N, R)
    w16, gidx = _make_weights(N, E)(dst, edge_type, src, wtab)

    edge_k = _make_edgepass(N, E, (R + 1) * N)

    wall1 = jnp.concatenate([W1, root1[None]], axis=0)
    ball1 = jnp.zeros((R + 1, 1, H), jnp.float32).at[R, 0].set(b1)
    yall1 = _transform(x, wall1, ball1)
    p1 = edge_k(yall1, gidx, dst, w16)
    h1 = _combine(yall1, p1, N, H, R)

    wall2 = jnp.concatenate([W2, root2[None]], axis=0)
    ball2 = jnp.zeros((R + 1, 1, H), jnp.float32).at[R, 0].set(b2)
    yall2 = _transform(h1, wall2, ball2)
    p2 = edge_k(yall2, gidx, dst, w16)
    h2 = _combine(yall2, p2, N, H, R)

    logits, probs = _mlp(h2, mlp1_w, mlp1_b, mlp2_w, mlp2_b)
    return (logits, probs, h2)


# speedup vs baseline: 12.9683x; 1.5130x over previous
import functools

import jax
import jax.numpy as jnp
from jax import lax
from jax.experimental import pallas as pl
from jax.experimental.pallas import tpu as pltpu
from jax.experimental.pallas import tpu_sc as plsc

_NC = 2
_NS = 16
_NW = _NC * _NS
_CH = 80
_ZCH = 104
_BN = 1000


def _sc_mesh():
    return plsc.VectorSubcoreMesh(core_axis_name="c", subcore_axis_name="s")


@functools.lru_cache(maxsize=None)
def _make_counts(N, E, R):
    EW = E // _NW
    nchunk = EW // _CH
    rpt = (N // _NS) // 8 * 8
    extra = N - _NS * rpt
    nz = rpt // _ZCH
    assert rpt % _ZCH == 0 and extra % 8 == 0 and extra <= _ZCH

    @functools.partial(
        pl.kernel, mesh=_sc_mesh(),
        out_type=jax.ShapeDtypeStruct((_NC, N, 128), jnp.float32),
        scratch_types=[
            pltpu.VMEM((_CH,), jnp.int32),
            pltpu.VMEM((_CH,), jnp.int32),
            pltpu.VMEM((_CH, 128), jnp.float32),
            pltpu.VMEM((_ZCH, 128), jnp.float32),
            pltpu.VMEM_SHARED((8, 128), jnp.float32),
            pltpu.VMEM_SHARED((N, 128), jnp.float32),
            pltpu.SemaphoreType.DMA,
        ],
    )
    def counts_k(dst, etype, onehot, cntp,
                 dst_v, typ_v, oh_v, z_v, ohs, cnt2, sem):
        cid = lax.axis_index("c")
        sid = lax.axis_index("s")
        wid = cid * _NS + sid

        def zrow(j, c):
            for k in range(8):
                z_v[j, pl.ds(k * 16, 16)] = jnp.zeros((16,), jnp.float32)
            return c
        lax.fori_loop(0, _ZCH, zrow, 0)
        base_r = pl.multiple_of(sid * rpt, 8)
        for m in range(nz):
            pltpu.sync_copy(z_v, cnt2.at[pl.ds(base_r + m * _ZCH, _ZCH)])
        if extra:
            @pl.when(sid == _NS - 1)
            def _():
                pltpu.sync_copy(z_v.at[pl.ds(0, extra)],
                                cnt2.at[pl.ds(_NS * rpt, extra)])

        @pl.when(sid == 0)
        def _():
            pltpu.sync_copy(onehot, ohs)
        plsc.subcore_barrier()

        ebase = wid * EW

        def chunk(c, carry):
            off = pl.multiple_of(ebase + c * _CH, 8)
            pltpu.sync_copy(dst.at[pl.ds(off, _CH)], dst_v)
            pltpu.sync_copy(etype.at[pl.ds(off, _CH)], typ_v)
            pltpu.async_copy(ohs.at[typ_v], oh_v, sem).wait()
            pltpu.sync_copy(oh_v, cnt2.at[dst_v], add=True)
            return carry
        lax.fori_loop(0, nchunk, chunk, 0)
        plsc.subcore_barrier()
        pltpu.sync_copy(cnt2.at[pl.ds(base_r, rpt)],
                        cntp.at[cid, pl.ds(base_r, rpt)])
        if extra:
            @pl.when(sid == _NS - 1)
            def _():
                pltpu.sync_copy(cnt2.at[pl.ds(_NS * rpt, extra)],
                                cntp.at[cid, pl.ds(_NS * rpt, extra)])

    return counts_k


def _weight_table(cntp, mhot, N, R):
    NB = N // _BN

    def body(p_ref, m_ref, o_ref):
        w = 1.0 / jnp.maximum(p_ref[0] + p_ref[1], 1.0)
        o_ref[...] = jnp.dot(w, m_ref[0], preferred_element_type=jnp.float32)

    return pl.pallas_call(
        body,
        grid=(R, NB),
        in_specs=[
            pl.BlockSpec((2, _BN, 128), lambda t, i: (0, i, 0)),
            pl.BlockSpec((1, 128, 128), lambda t, i: (t, 0, 0)),
        ],
        out_specs=pl.BlockSpec((_BN, 128), lambda t, i: (t * NB + i, 0)),
        out_shape=jax.ShapeDtypeStruct((R * N, 128), jnp.float32),
    )(cntp, mhot)


@functools.lru_cache(maxsize=None)
def _make_weights(N, E):
    EW = E // _NW
    nchunk = EW // _CH
    assert nchunk % 2 == 1

    @functools.partial(
        pl.kernel, mesh=_sc_mesh(),
        out_type=(jax.ShapeDtypeStruct((E, 16), jnp.float32),
                  jax.ShapeDtypeStruct((E,), jnp.int32)),
        scratch_types=[
            pltpu.VMEM((_CH,), jnp.int32),
            pltpu.VMEM((_CH,), jnp.int32),
            pltpu.VMEM((2, _CH), jnp.int32),
            pltpu.VMEM((2, _CH), jnp.int32),
            pltpu.VMEM((2, _CH, 128), jnp.float32),
            pltpu.VMEM((_CH, 16), jnp.float32),
            pltpu.SemaphoreType.DMA((2,)),
        ],
    )
    def weights_k(dst, etype, srcv, wtab, w16, gidx,
                  dst_v, typ_v, src_2, g2_2, ra_2, w_v, sem):
        cid = lax.axis_index("c")
        sid = lax.axis_index("s")
        wid = cid * _NS + sid
        ebase = wid * EW

        def issue(c, b):
            off = pl.multiple_of(ebase + c * _CH, 8)
            pltpu.sync_copy(dst.at[pl.ds(off, _CH)], dst_v)
            pltpu.sync_copy(etype.at[pl.ds(off, _CH)], typ_v)
            pltpu.sync_copy(srcv.at[pl.ds(off, _CH)], src_2.at[b])
            for g in range(_CH // 16):
                t = typ_v[pl.ds(g * 16, 16)]
                g2_2[b, pl.ds(g * 16, 16)] = t * N + dst_v[pl.ds(g * 16, 16)]
                src_2[b, pl.ds(g * 16, 16)] = (t * N
                                               + src_2[b, pl.ds(g * 16, 16)])
            pltpu.async_copy(wtab.at[g2_2.at[b]], ra_2.at[b], sem.at[b])

        def step(c, b, issue_next):
            if issue_next:
                issue(c + 1, 1 - b)
            off = pl.multiple_of(ebase + c * _CH, 8)
            pltpu.make_async_copy(
                wtab.at[g2_2.at[b]], ra_2.at[b], sem.at[b]).wait()

            def ext(j, cc):
                w_v[j, pl.ds(0, 16)] = ra_2[b, j, pl.ds(0, 16)]
                return cc
            lax.fori_loop(0, _CH, ext, 0)
            pltpu.sync_copy(w_v, w16.at[pl.ds(off, _CH)])
            pltpu.sync_copy(src_2.at[b], gidx.at[pl.ds(off, _CH)])

        issue(0, 0)

        def pair(c0, carry):
            step(c0 * 2, 0, True)
            step(c0 * 2 + 1, 1, True)
            return carry
        lax.fori_loop(0, nchunk // 2, pair, 0)
        if nchunk % 2:
            step(nchunk - 1, 0, False)

    return weights_k


@functools.lru_cache(maxsize=None)
def _make_edgepass(N, E, RN1):
    EW = E // _NW
    nchunk = EW // _CH
    rpt = (N // _NS) // 8 * 8
    extra = N - _NS * rpt
    nzf = rpt // _CH
    nzr = rpt - nzf * _CH
    assert nchunk % 2 == 1 and nzr % 8 == 0 and extra % 8 == 0 and extra <= _CH

    @functools.partial(
        pl.kernel, mesh=_sc_mesh(),
        out_type=jax.ShapeDtypeStruct((_NC, N, 128), jnp.float32),
        scratch_types=[
            pltpu.VMEM((2, _CH), jnp.int32),
            pltpu.VMEM((2, _CH), jnp.int32),
            pltpu.VMEM((2, _CH, 16), jnp.float32),
            pltpu.VMEM((2, _CH, 128), jnp.float32),
            pltpu.VMEM_SHARED((N, 128), jnp.float32),
            pltpu.SemaphoreType.DMA((2,)),
            pltpu.SemaphoreType.DMA((2,)),
        ],
    )
    def edge_k(yall, gidx, dst, w16, outp,
               gi_2, dst_2, w_2, rows_2, accum, sem, wsem):
        cid = lax.axis_index("c")
        sid = lax.axis_index("s")
        wid = cid * _NS + sid

        def zrow(j, c):
            for k in range(8):
                rows_2[0, j, pl.ds(k * 16, 16)] = jnp.zeros((16,), jnp.float32)
            return c
        lax.fori_loop(0, _CH, zrow, 0)
        base_r = pl.multiple_of(sid * rpt, 8)
        for m in range(nzf):
            pltpu.sync_copy(rows_2.at[0],
                            accum.at[pl.ds(base_r + m * _CH, _CH)])
        if nzr:
            pltpu.sync_copy(rows_2.at[0, pl.ds(0, nzr)],
                            accum.at[pl.ds(base_r + nzf * _CH, nzr)])
        if extra:
            @pl.when(sid == _NS - 1)
            def _():
                pltpu.sync_copy(rows_2.at[0, pl.ds(0, extra)],
                                accum.at[pl.ds(_NS * rpt, extra)])
        plsc.subcore_barrier()

        ebase = wid * EW

        def issue(c, b):
            off = pl.multiple_of(ebase + c * _CH, 8)
            pltpu.sync_copy(gidx.at[pl.ds(off, _CH)], gi_2.at[b])
            pltpu.sync_copy(dst.at[pl.ds(off, _CH)], dst_2.at[b])
            pltpu.async_copy(w16.at[pl.ds(off, _CH)], w_2.at[b], wsem.at[b])
            pltpu.async_copy(yall.at[gi_2.at[b]], rows_2.at[b], sem.at[b])

        def step(c, b, issue_next):
            if issue_next:
                issue(c + 1, 1 - b)
            off = pl.multiple_of(ebase + c * _CH, 8)
            pltpu.make_async_copy(
                yall.at[gi_2.at[b]], rows_2.at[b], sem.at[b]).wait()
            pltpu.make_async_copy(
                w16.at[pl.ds(off, _CH)], w_2.at[b], wsem.at[b]).wait()

            def srow(j, cc):
                wj = w_2[b, j, pl.ds(0, 16)]
                for k in range(8):
                    rows_2[b, j, pl.ds(k * 16, 16)] = (
                        rows_2[b, j, pl.ds(k * 16, 16)] * wj)
                return cc
            lax.fori_loop(0, _CH, srow, 0)
            pltpu.sync_copy(rows_2.at[b], accum.at[dst_2.at[b]], add=True)

        issue(0, 0)

        def pair(c0, carry):
            step(c0 * 2, 0, True)
            step(c0 * 2 + 1, 1, True)
            return carry
        lax.fori_loop(0, nchunk // 2, pair, 0)
        step(nchunk - 1, 0, False)
        plsc.subcore_barrier()
        pltpu.sync_copy(accum.at[pl.ds(base_r, rpt)],
                        outp.at[cid, pl.ds(base_r, rpt)])
        if extra:
            @pl.when(sid == _NS - 1)
            def _():
                pltpu.sync_copy(accum.at[pl.ds(_NS * rpt, extra)],
                                outp.at[cid, pl.ds(_NS * rpt, extra)])

    return edge_k


def _transform(x, wall, ball):
    N, D = x.shape
    R1, _, H = wall.shape
    NB = N // _BN

    def body(x_ref, w_ref, b_ref, o_ref):
        o_ref[...] = (jnp.dot(x_ref[...], w_ref[0],
                              preferred_element_type=jnp.float32) + b_ref[0])

    return pl.pallas_call(
        body,
        grid=(R1, NB),
        in_specs=[
            pl.BlockSpec((_BN, D), lambda r, i: (i, 0)),
            pl.BlockSpec((1, D, H), lambda r, i: (r, 0, 0)),
            pl.BlockSpec((1, 1, H), lambda r, i: (r, 0, 0)),
        ],
        out_specs=pl.BlockSpec((_BN, H), lambda r, i: (r * NB + i, 0)),
        out_shape=jax.ShapeDtypeStruct((R1 * N, H), jnp.float32),
    )(x, wall, ball)


def _combine(yall, partials, N, H, R):
    NB = N // _BN

    def body(y_ref, p_ref, o_ref):
        o_ref[...] = jnp.maximum(y_ref[...] + p_ref[0] + p_ref[1], 0.0)

    return pl.pallas_call(
        body,
        grid=(NB,),
        in_specs=[
            pl.BlockSpec((_BN, H), lambda i: (R * NB + i, 0)),
            pl.BlockSpec((2, _BN, H), lambda i: (0, i, 0)),
        ],
        out_specs=pl.BlockSpec((_BN, H), lambda i: (i, 0)),
        out_shape=jax.ShapeDtypeStruct((N, H), jnp.float32),
    )(yall, partials)


def _mlp(h, w1, b1, w2, b2):
    N, H = h.shape
    C = w2.shape[1]
    NB = N // _BN
    b1r = b1.reshape(1, H)
    b2r = b2.reshape(1, C)

    def body(h_ref, w1_ref, b1_ref, w2_ref, b2_ref, lo_ref, po_ref):
        z = (jnp.dot(h_ref[...], w1_ref[...],
                     preferred_element_type=jnp.float32) + b1_ref[...])
        z = jnp.where(z > 0, z, jnp.exp(jnp.minimum(z, 0.0)) - 1.0)
        lg = (jnp.dot(z, w2_ref[...],
                      preferred_element_type=jnp.float32) + b2_ref[...])
        lo_ref[...] = lg
        m = jnp.max(lg, axis=-1, keepdims=True)
        e = jnp.exp(lg - m)
        po_ref[...] = e / jnp.sum(e, axis=-1, keepdims=True)

    return pl.pallas_call(
        body,
        grid=(NB,),
        in_specs=[
            pl.BlockSpec((_BN, H), lambda i: (i, 0)),
            pl.BlockSpec((H, H), lambda i: (0, 0)),
            pl.BlockSpec((1, H), lambda i: (0, 0)),
            pl.BlockSpec((H, C), lambda i: (0, 0)),
            pl.BlockSpec((1, C), lambda i: (0, 0)),
        ],
        out_specs=(pl.BlockSpec((_BN, C), lambda i: (i, 0)),
                   pl.BlockSpec((_BN, C), lambda i: (i, 0))),
        out_shape=(jax.ShapeDtypeStruct((N, C), jnp.float32),
                   jax.ShapeDtypeStruct((N, C), jnp.float32)),
    )(h, w1, b1r, w2, b2r)


def kernel(x, edge_index, edge_type, W1, root1, b1, W2, root2, b2,
           mlp1_w, mlp1_b, mlp2_w, mlp2_b):
    N, D = x.shape
    E = edge_type.shape[0]
    R = W1.shape[0]
    H = root1.shape[1]
    assert D == 128 and H == 128 and E % (_NW * _CH) == 0 and N % _NS == 0

    src = edge_index[0]
    dst = edge_index[1]
    onehot = (jax.lax.broadcasted_iota(jnp.int32, (R, 128), 1) // 16
              == jax.lax.broadcasted_iota(jnp.int32, (R, 128), 0)
              ).astype(jnp.float32)
    mhot = (jax.lax.broadcasted_iota(jnp.int32, (R, 128, 128), 1)
            == 16 * jax.lax.broadcasted_iota(jnp.int32, (R, 128, 128), 0)
            + jax.lax.broadcasted_iota(jnp.int32, (R, 128, 128), 2) % 16
            ).astype(jnp.float32)

    cntp = _make_counts(N, E, R)(dst, edge_type, onehot)
    wtab = _weight_table(cntp, mhot, N, R)
    w16, gidx = _make_weights(N, E)(dst, edge_type, src, wtab)

    edge_k = _make_edgepass(N, E, (R + 1) * N)

    wall1 = jnp.concatenate([W1, root1[None]], axis=0)
    ball1 = jnp.zeros((R + 1, 1, H), jnp.float32).at[R, 0].set(b1)
    yall1 = _transform(x, wall1, ball1)
    p1 = edge_k(yall1, gidx, dst, w16)
    h1 = _combine(yall1, p1, N, H, R)

    wall2 = jnp.concatenate([W2, root2[None]], axis=0)
    ball2 = jnp.zeros((R + 1, 1, H), jnp.float32).at[R, 0].set(b2)
    yall2 = _transform(h1, wall2, ball2)
    p2 = edge_k(yall2, gidx, dst, w16)
    h2 = _combine(yall2, p2, N, H, R)

    logits, probs = _mlp(h2, mlp1_w, mlp1_b, mlp2_w, mlp2_b)
    return (logits, probs, h2)


# speedup vs baseline: 13.2856x; 1.0245x over previous
import functools

import jax
import jax.numpy as jnp
from jax import lax
from jax.experimental import pallas as pl
from jax.experimental.pallas import tpu as pltpu
from jax.experimental.pallas import tpu_sc as plsc

_NC = 2
_NS = 16
_NW = _NC * _NS
_CH = 80
_ZCH = 104
_BN = 1000


def _sc_mesh():
    return plsc.VectorSubcoreMesh(core_axis_name="c", subcore_axis_name="s")


@functools.lru_cache(maxsize=None)
def _make_counts(N, E, R):
    EW = E // _NW
    nchunk = EW // _CH
    rpt = (N // _NS) // 8 * 8
    extra = N - _NS * rpt
    nzf = rpt // _CH
    nzr = rpt - nzf * _CH
    assert nchunk % 2 == 1 and nzr % 8 == 0 and extra % 8 == 0 and extra <= _CH

    @functools.partial(
        pl.kernel, mesh=_sc_mesh(),
        out_type=jax.ShapeDtypeStruct((_NC, N, 128), jnp.float32),
        scratch_types=[
            pltpu.VMEM((2, _CH), jnp.int32),
            pltpu.VMEM((2, _CH), jnp.int32),
            pltpu.VMEM((2, _CH, 128), jnp.float32),
            pltpu.VMEM_SHARED((8, 128), jnp.float32),
            pltpu.VMEM_SHARED((N, 128), jnp.float32),
            pltpu.SemaphoreType.DMA((2,)),
        ],
    )
    def counts_k(dst, etype, onehot, cntp,
                 dst_2, typ_2, oh_2, ohs, cnt2, sem):
        cid = lax.axis_index("c")
        sid = lax.axis_index("s")
        wid = cid * _NS + sid

        def zrow(j, c):
            for k in range(8):
                oh_2[0, j, pl.ds(k * 16, 16)] = jnp.zeros((16,), jnp.float32)
            return c
        lax.fori_loop(0, _CH, zrow, 0)
        base_r = pl.multiple_of(sid * rpt, 8)
        for m in range(nzf):
            pltpu.sync_copy(oh_2.at[0], cnt2.at[pl.ds(base_r + m * _CH, _CH)])
        if nzr:
            pltpu.sync_copy(oh_2.at[0, pl.ds(0, nzr)],
                            cnt2.at[pl.ds(base_r + nzf * _CH, nzr)])
        if extra:
            @pl.when(sid == _NS - 1)
            def _():
                pltpu.sync_copy(oh_2.at[0, pl.ds(0, extra)],
                                cnt2.at[pl.ds(_NS * rpt, extra)])

        @pl.when(sid == 0)
        def _():
            pltpu.sync_copy(onehot, ohs)
        plsc.subcore_barrier()

        ebase = wid * EW

        def issue(c, b):
            off = pl.multiple_of(ebase + c * _CH, 8)
            pltpu.sync_copy(dst.at[pl.ds(off, _CH)], dst_2.at[b])
            pltpu.sync_copy(etype.at[pl.ds(off, _CH)], typ_2.at[b])
            pltpu.async_copy(ohs.at[typ_2.at[b]], oh_2.at[b], sem.at[b])

        def step(c, b, issue_next):
            if issue_next:
                issue(c + 1, 1 - b)
            pltpu.make_async_copy(
                ohs.at[typ_2.at[b]], oh_2.at[b], sem.at[b]).wait()
            pltpu.sync_copy(oh_2.at[b], cnt2.at[dst_2.at[b]], add=True)

        issue(0, 0)

        def pair(c0, carry):
            step(c0 * 2, 0, True)
            step(c0 * 2 + 1, 1, True)
            return carry
        lax.fori_loop(0, nchunk // 2, pair, 0)
        step(nchunk - 1, 0, False)
        plsc.subcore_barrier()
        pltpu.sync_copy(cnt2.at[pl.ds(base_r, rpt)],
                        cntp.at[cid, pl.ds(base_r, rpt)])
        if extra:
            @pl.when(sid == _NS - 1)
            def _():
                pltpu.sync_copy(cnt2.at[pl.ds(_NS * rpt, extra)],
                                cntp.at[cid, pl.ds(_NS * rpt, extra)])

    return counts_k


def _weight_table(cntp, mhot, N, R):
    NB = N // _BN

    def body(p_ref, m_ref, o_ref):
        w = 1.0 / jnp.maximum(p_ref[0] + p_ref[1], 1.0)
        o_ref[...] = jnp.dot(w, m_ref[0], preferred_element_type=jnp.float32)

    return pl.pallas_call(
        body,
        grid=(R, NB),
        in_specs=[
            pl.BlockSpec((2, _BN, 128), lambda t, i: (0, i, 0)),
            pl.BlockSpec((1, 128, 128), lambda t, i: (t, 0, 0)),
        ],
        out_specs=pl.BlockSpec((_BN, 128), lambda t, i: (t * NB + i, 0)),
        out_shape=jax.ShapeDtypeStruct((R * N, 128), jnp.float32),
    )(cntp, mhot)


@functools.lru_cache(maxsize=None)
def _make_weights(N, E):
    EW = E // _NW
    nchunk = EW // _CH
    assert nchunk % 2 == 1

    @functools.partial(
        pl.kernel, mesh=_sc_mesh(),
        out_type=(jax.ShapeDtypeStruct((E, 16), jnp.float32),
                  jax.ShapeDtypeStruct((E,), jnp.int32)),
        scratch_types=[
            pltpu.VMEM((_CH,), jnp.int32),
            pltpu.VMEM((_CH,), jnp.int32),
            pltpu.VMEM((2, _CH), jnp.int32),
            pltpu.VMEM((2, _CH), jnp.int32),
            pltpu.VMEM((2, _CH, 128), jnp.float32),
            pltpu.VMEM((_CH, 16), jnp.float32),
            pltpu.SemaphoreType.DMA((2,)),
        ],
    )
    def weights_k(dst, etype, srcv, wtab, w16, gidx,
                  dst_v, typ_v, src_2, g2_2, ra_2, w_v, sem):
        cid = lax.axis_index("c")
        sid = lax.axis_index("s")
        wid = cid * _NS + sid
        ebase = wid * EW

        def issue(c, b):
            off = pl.multiple_of(ebase + c * _CH, 8)
            pltpu.sync_copy(dst.at[pl.ds(off, _CH)], dst_v)
            pltpu.sync_copy(etype.at[pl.ds(off, _CH)], typ_v)
            pltpu.sync_copy(srcv.at[pl.ds(off, _CH)], src_2.at[b])
            for g in range(_CH // 16):
                t = typ_v[pl.ds(g * 16, 16)]
                g2_2[b, pl.ds(g * 16, 16)] = t * N + dst_v[pl.ds(g * 16, 16)]
                src_2[b, pl.ds(g * 16, 16)] = (t * N
                                               + src_2[b, pl.ds(g * 16, 16)])
            pltpu.async_copy(wtab.at[g2_2.at[b]], ra_2.at[b], sem.at[b])

        def step(c, b, issue_next):
            if issue_next:
                issue(c + 1, 1 - b)
            off = pl.multiple_of(ebase + c * _CH, 8)
            pltpu.make_async_copy(
                wtab.at[g2_2.at[b]], ra_2.at[b], sem.at[b]).wait()

            def ext(j, cc):
                w_v[j, pl.ds(0, 16)] = ra_2[b, j, pl.ds(0, 16)]
                return cc
            lax.fori_loop(0, _CH, ext, 0)
            pltpu.sync_copy(w_v, w16.at[pl.ds(off, _CH)])
            pltpu.sync_copy(src_2.at[b], gidx.at[pl.ds(off, _CH)])

        issue(0, 0)

        def pair(c0, carry):
            step(c0 * 2, 0, True)
            step(c0 * 2 + 1, 1, True)
            return carry
        lax.fori_loop(0, nchunk // 2, pair, 0)
        if nchunk % 2:
            step(nchunk - 1, 0, False)

    return weights_k


@functools.lru_cache(maxsize=None)
def _make_edgepass(N, E, RN1):
    EW = E // _NW
    nchunk = EW // _CH
    rpt = (N // _NS) // 8 * 8
    extra = N - _NS * rpt
    nzf = rpt // _CH
    nzr = rpt - nzf * _CH
    assert nchunk % 2 == 1 and nzr % 8 == 0 and extra % 8 == 0 and extra <= _CH

    @functools.partial(
        pl.kernel, mesh=_sc_mesh(),
        out_type=jax.ShapeDtypeStruct((_NC, N, 128), jnp.float32),
        scratch_types=[
            pltpu.VMEM((2, _CH), jnp.int32),
            pltpu.VMEM((2, _CH), jnp.int32),
            pltpu.VMEM((2, _CH, 16), jnp.float32),
            pltpu.VMEM((2, _CH, 128), jnp.float32),
            pltpu.VMEM_SHARED((N, 128), jnp.float32),
            pltpu.SemaphoreType.DMA((2,)),
            pltpu.SemaphoreType.DMA((2,)),
        ],
    )
    def edge_k(yall, gidx, dst, w16, outp,
               gi_2, dst_2, w_2, rows_2, accum, sem, wsem):
        cid = lax.axis_index("c")
        sid = lax.axis_index("s")
        wid = cid * _NS + sid

        def zrow(j, c):
            for k in range(8):
                rows_2[0, j, pl.ds(k * 16, 16)] = jnp.zeros((16,), jnp.float32)
            return c
        lax.fori_loop(0, _CH, zrow, 0)
        base_r = pl.multiple_of(sid * rpt, 8)
        for m in range(nzf):
            pltpu.sync_copy(rows_2.at[0],
                            accum.at[pl.ds(base_r + m * _CH, _CH)])
        if nzr:
            pltpu.sync_copy(rows_2.at[0, pl.ds(0, nzr)],
                            accum.at[pl.ds(base_r + nzf * _CH, nzr)])
        if extra:
            @pl.when(sid == _NS - 1)
            def _():
                pltpu.sync_copy(rows_2.at[0, pl.ds(0, extra)],
                                accum.at[pl.ds(_NS * rpt, extra)])
        plsc.subcore_barrier()

        ebase = wid * EW

        def issue(c, b):
            off = pl.multiple_of(ebase + c * _CH, 8)
            pltpu.sync_copy(gidx.at[pl.ds(off, _CH)], gi_2.at[b])
            pltpu.sync_copy(dst.at[pl.ds(off, _CH)], dst_2.at[b])
            pltpu.async_copy(w16.at[pl.ds(off, _CH)], w_2.at[b], wsem.at[b])
            pltpu.async_copy(yall.at[gi_2.at[b]], rows_2.at[b], sem.at[b])

        def step(c, b, issue_next):
            if issue_next:
                issue(c + 1, 1 - b)
            off = pl.multiple_of(ebase + c * _CH, 8)
            pltpu.make_async_copy(
                yall.at[gi_2.at[b]], rows_2.at[b], sem.at[b]).wait()
            pltpu.make_async_copy(
                w16.at[pl.ds(off, _CH)], w_2.at[b], wsem.at[b]).wait()

            def srow(j, cc):
                wj = w_2[b, j, pl.ds(0, 16)]
                for k in range(8):
                    rows_2[b, j, pl.ds(k * 16, 16)] = (
                        rows_2[b, j, pl.ds(k * 16, 16)] * wj)
                return cc
            lax.fori_loop(0, _CH, srow, 0)
            pltpu.sync_copy(rows_2.at[b], accum.at[dst_2.at[b]], add=True)

        issue(0, 0)

        def pair(c0, carry):
            step(c0 * 2, 0, True)
            step(c0 * 2 + 1, 1, True)
            return carry
        lax.fori_loop(0, nchunk // 2, pair, 0)
        step(nchunk - 1, 0, False)
        plsc.subcore_barrier()
        pltpu.sync_copy(accum.at[pl.ds(base_r, rpt)],
                        outp.at[cid, pl.ds(base_r, rpt)])
        if extra:
            @pl.when(sid == _NS - 1)
            def _():
                pltpu.sync_copy(accum.at[pl.ds(_NS * rpt, extra)],
                                outp.at[cid, pl.ds(_NS * rpt, extra)])

    return edge_k


def _transform(x, wall, ball):
    N, D = x.shape
    R1, _, H = wall.shape
    NB = N // _BN

    def body(x_ref, w_ref, b_ref, o_ref):
        o_ref[...] = (jnp.dot(x_ref[...], w_ref[0],
                              preferred_element_type=jnp.float32) + b_ref[0])

    return pl.pallas_call(
        body,
        grid=(R1, NB),
        in_specs=[
            pl.BlockSpec((_BN, D), lambda r, i: (i, 0)),
            pl.BlockSpec((1, D, H), lambda r, i: (r, 0, 0)),
            pl.BlockSpec((1, 1, H), lambda r, i: (r, 0, 0)),
        ],
        out_specs=pl.BlockSpec((_BN, H), lambda r, i: (r * NB + i, 0)),
        out_shape=jax.ShapeDtypeStruct((R1 * N, H), jnp.float32),
    )(x, wall, ball)


def _combine_transform(yall, partials, wall, ball, N, H, R):
    R1 = wall.shape[0]
    NB = N // _BN

    def body(y_ref, p_ref, w_ref, b_ref, o_ref):
        h = jnp.maximum(y_ref[...] + p_ref[0] + p_ref[1], 0.0)
        o_ref[...] = (jnp.dot(h, w_ref[0],
                              preferred_element_type=jnp.float32) + b_ref[0])

    return pl.pallas_call(
        body,
        grid=(R1, NB),
        in_specs=[
            pl.BlockSpec((_BN, H), lambda r, i: (R * NB + i, 0)),
            pl.BlockSpec((2, _BN, H), lambda r, i: (0, i, 0)),
            pl.BlockSpec((1, H, H), lambda r, i: (r, 0, 0)),
            pl.BlockSpec((1, 1, H), lambda r, i: (r, 0, 0)),
        ],
        out_specs=pl.BlockSpec((_BN, H), lambda r, i: (r * NB + i, 0)),
        out_shape=jax.ShapeDtypeStruct((R1 * N, H), jnp.float32),
    )(yall, partials, wall, ball)


def _combine_mlp(yall, partials, w1, b1, w2, b2, N, H, R):
    C = w2.shape[1]
    NB = N // _BN
    b1r = b1.reshape(1, H)
    b2r = b2.reshape(1, C)

    def body(y_ref, p_ref, w1_ref, b1_ref, w2_ref, b2_ref,
             h_ref, lo_ref, po_ref):
        h = jnp.maximum(y_ref[...] + p_ref[0] + p_ref[1], 0.0)
        h_ref[...] = h
        z = (jnp.dot(h, w1_ref[...],
                     preferred_element_type=jnp.float32) + b1_ref[...])
        z = jnp.where(z > 0, z, jnp.exp(jnp.minimum(z, 0.0)) - 1.0)
        lg = (jnp.dot(z, w2_ref[...],
                      preferred_element_type=jnp.float32) + b2_ref[...])
        lo_ref[...] = lg
        m = jnp.max(lg, axis=-1, keepdims=True)
        e = jnp.exp(lg - m)
        po_ref[...] = e / jnp.sum(e, axis=-1, keepdims=True)

    return pl.pallas_call(
        body,
        grid=(NB,),
        in_specs=[
            pl.BlockSpec((_BN, H), lambda i: (R * NB + i, 0)),
            pl.BlockSpec((2, _BN, H), lambda i: (0, i, 0)),
            pl.BlockSpec((H, H), lambda i: (0, 0)),
            pl.BlockSpec((1, H), lambda i: (0, 0)),
            pl.BlockSpec((H, C), lambda i: (0, 0)),
            pl.BlockSpec((1, C), lambda i: (0, 0)),
        ],
        out_specs=(pl.BlockSpec((_BN, H), lambda i: (i, 0)),
                   pl.BlockSpec((_BN, C), lambda i: (i, 0)),
                   pl.BlockSpec((_BN, C), lambda i: (i, 0))),
        out_shape=(jax.ShapeDtypeStruct((N, H), jnp.float32),
                   jax.ShapeDtypeStruct((N, C), jnp.float32),
                   jax.ShapeDtypeStruct((N, C), jnp.float32)),
    )(yall, partials, w1, b1r, w2, b2r)


def kernel(x, edge_index, edge_type, W1, root1, b1, W2, root2, b2,
           mlp1_w, mlp1_b, mlp2_w, mlp2_b):
    N, D = x.shape
    E = edge_type.shape[0]
    R = W1.shape[0]
    H = root1.shape[1]
    assert D == 128 and H == 128 and E % (_NW * _CH) == 0 and N % _NS == 0

    src = edge_index[0]
    dst = edge_index[1]
    onehot = (jax.lax.broadcasted_iota(jnp.int32, (R, 128), 1) // 16
              == jax.lax.broadcasted_iota(jnp.int32, (R, 128), 0)
              ).astype(jnp.float32)
    mhot = (jax.lax.broadcasted_iota(jnp.int32, (R, 128, 128), 1)
            == 16 * jax.lax.broadcasted_iota(jnp.int32, (R, 128, 128), 0)
            + jax.lax.broadcasted_iota(jnp.int32, (R, 128, 128), 2) % 16
            ).astype(jnp.float32)

    cntp = _make_counts(N, E, R)(dst, edge_type, onehot)
    wtab = _weight_table(cntp, mhot, N, R)
    w16, gidx = _make_weights(N, E)(dst, edge_type, src, wtab)

    edge_k = _make_edgepass(N, E, (R + 1) * N)

    wall1 = jnp.concatenate([W1, root1[None]], axis=0)
    ball1 = jnp.zeros((R + 1, 1, H), jnp.float32).at[R, 0].set(b1)
    yall1 = _transform(x, wall1, ball1)
    p1 = edge_k(yall1, gidx, dst, w16)

    wall2 = jnp.concatenate([W2, root2[None]], axis=0)
    ball2 = jnp.zeros((R + 1, 1, H), jnp.float32).at[R, 0].set(b2)
    yall2 = _combine_transform(yall1, p1, wall2, ball2, N, H, R)
    p2 = edge_k(yall2, gidx, dst, w16)

    h2, logits, probs = _combine_mlp(yall2, p2, mlp1_w, mlp1_b,
                                     mlp2_w, mlp2_b, N, H, R)
    return (logits, probs, h2)
